# Initial kernel scaffold; baseline (speedup 1.0000x reference)
import jax
import jax.numpy as jnp
from jax.experimental import pallas as pl


def kernel(x, f_ij, idx_i, idx_j, rcut_ij, W_in, b_in, Wf1, bf1, Wf2, bf2, Wo1, bo1, Wo2, bo2):
    raise NotImplementedError("write your pallas kernel here")


# speedup vs baseline: 1.9035x; 1.9035x over previous
import functools

import jax
import jax.numpy as jnp
from jax import lax
from jax.experimental import pallas as pl
from jax.experimental.pallas import tpu as pltpu
from jax.experimental.pallas import tpu_sc as plsc

N_ATOMS = 10000
D = 128
N_RBF = 20
N_PAIRS = 320000

NC, NS = 2, 16
NW = NC * NS
K = 128
CHUNKS = -(-N_PAIRS // (NW * K))
NPAD = CHUNKS * NW * K
PAIRS_PER_W = CHUNKS * K
N_ROWS_PAD = 10240
ROWS_PER_TILE = N_ROWS_PAD // NS

_LOG2 = 0.6931471805599453


def _ssp(t):
    return jnp.maximum(t, 0.0) + jnp.log1p(jnp.exp(-jnp.abs(t))) - _LOG2


P_BLK = 4096


def _tc_a_body(x_ref, W_in_ref, b_in_ref, f_ref, rc_ref, Wf1_ref, bf1_ref,
               Wf2_ref, bf2_ref, h_ref, wij_ref):
    @pl.when(pl.program_id(0) == 0)
    def _():
        h_ref[...] = lax.dot_general(
            x_ref[...], W_in_ref[...], (((1,), (1,)), ((), ())),
            preferred_element_type=jnp.float32) + b_in_ref[...]

    t = lax.dot_general(f_ref[...], Wf1_ref[...], (((1,), (1,)), ((), ())),
                        preferred_element_type=jnp.float32) + bf1_ref[...]
    w = lax.dot_general(_ssp(t), Wf2_ref[...], (((1,), (1,)), ((), ())),
                        preferred_element_type=jnp.float32) + bf2_ref[...]
    wij_ref[...] = w * rc_ref[...]


def _tc_a(x2, W_in, b_in, f_pad, rc_pad, Wf1, bf1, Wf2, bf2):
    nblk = NPAD // P_BLK
    return pl.pallas_call(
        _tc_a_body,
        grid=(nblk,),
        in_specs=[
            pl.BlockSpec((N_ATOMS, D), lambda i: (0, 0)),
            pl.BlockSpec((D, D), lambda i: (0, 0)),
            pl.BlockSpec((1, D), lambda i: (0, 0)),
            pl.BlockSpec((P_BLK, N_RBF), lambda i: (i, 0)),
            pl.BlockSpec((P_BLK, 1), lambda i: (i, 0)),
            pl.BlockSpec((D, N_RBF), lambda i: (0, 0)),
            pl.BlockSpec((1, D), lambda i: (0, 0)),
            pl.BlockSpec((D, D), lambda i: (0, 0)),
            pl.BlockSpec((1, D), lambda i: (0, 0)),
        ],
        out_specs=[
            pl.BlockSpec((N_ATOMS, D), lambda i: (0, 0)),
            pl.BlockSpec((P_BLK, D), lambda i: (i, 0)),
        ],
        out_shape=[
            jax.ShapeDtypeStruct((N_ATOMS, D), jnp.float32),
            jax.ShapeDtypeStruct((NPAD, D), jnp.float32),
        ],
    )(x2, W_in, b_in, f_pad, rc_pad, Wf1, bf1, Wf2, bf2)


def _sc_body(h_hbm, wij_hbm, idxj_hbm, idxi_hbm, zeros_hbm, out_hbm,
             idxj_v, idxi_v, rows_v, wij_v, agg_sh, sem):
    c = lax.axis_index("c")
    s = lax.axis_index("s")
    wid = s * NC + c

    pltpu.sync_copy(zeros_hbm.at[pl.ds(s * ROWS_PER_TILE, ROWS_PER_TILE)],
                    agg_sh.at[pl.ds(s * ROWS_PER_TILE, ROWS_PER_TILE)])
    plsc.subcore_barrier()

    base0 = wid * PAIRS_PER_W

    def chunk(g, carry):
        base = base0 + g * K
        pltpu.sync_copy(idxj_hbm.at[pl.ds(base, K)], idxj_v)
        pltpu.sync_copy(idxi_hbm.at[pl.ds(base, K)], idxi_v)
        pltpu.sync_copy(wij_hbm.at[pl.ds(base, K)], wij_v)
        pltpu.async_copy(h_hbm.at[idxj_v], rows_v, sem).wait()

        def mul_row(r, carry2):
            for col in range(D // 16):
                sl = pl.ds(col * 16, 16)
                rows_v[r, sl] = rows_v[r, sl] * wij_v[r, sl]
            return carry2

        lax.fori_loop(0, K, mul_row, 0, unroll=False)
        pltpu.sync_copy(rows_v, agg_sh.at[idxi_v], add=True)
        return carry

    lax.fori_loop(0, CHUNKS, chunk, 0, unroll=False)
    plsc.subcore_barrier()

    pltpu.sync_copy(agg_sh.at[pl.ds(s * ROWS_PER_TILE, ROWS_PER_TILE)],
                    out_hbm.at[c, pl.ds(s * ROWS_PER_TILE, ROWS_PER_TILE)])


_sc_gather_scatter = functools.partial(
    pl.kernel,
    mesh=plsc.VectorSubcoreMesh(core_axis_name="c", subcore_axis_name="s"),
    out_type=jax.ShapeDtypeStruct((NC, N_ROWS_PAD, D), jnp.float32),
    scratch_types=[
        pltpu.VMEM((K,), jnp.int32),
        pltpu.VMEM((K,), jnp.int32),
        pltpu.VMEM((K, D), jnp.float32),
        pltpu.VMEM((K, D), jnp.float32),
        pltpu.VMEM_SHARED((N_ROWS_PAD, D), jnp.float32),
        pltpu.SemaphoreType.DMA,
    ],
)(_sc_body)


def _tc_b_body(p_ref, Wo1_ref, bo1_ref, Wo2_ref, bo2_ref, out_ref):
    agg = p_ref[0] + p_ref[1]
    t = lax.dot_general(agg, Wo1_ref[...], (((1,), (1,)), ((), ())),
                        preferred_element_type=jnp.float32) + bo1_ref[...]
    out_ref[...] = lax.dot_general(_ssp(t), Wo2_ref[...], (((1,), (1,)), ((), ())),
                                   preferred_element_type=jnp.float32) + bo2_ref[...]


def _tc_b(partials, Wo1, bo1, Wo2, bo2):
    return pl.pallas_call(
        _tc_b_body,
        grid=(1,),
        in_specs=[
            pl.BlockSpec((NC, N_ATOMS, D), lambda i: (0, 0, 0)),
            pl.BlockSpec((D, D), lambda i: (0, 0)),
            pl.BlockSpec((1, D), lambda i: (0, 0)),
            pl.BlockSpec((D, D), lambda i: (0, 0)),
            pl.BlockSpec((1, D), lambda i: (0, 0)),
        ],
        out_specs=pl.BlockSpec((N_ATOMS, D), lambda i: (0, 0)),
        out_shape=jax.ShapeDtypeStruct((N_ATOMS, D), jnp.float32),
    )(partials, Wo1, bo1, Wo2, bo2)


def kernel(x, f_ij, idx_i, idx_j, rcut_ij, W_in, b_in, Wf1, bf1, Wf2, bf2,
           Wo1, bo1, Wo2, bo2):
    x2 = x.reshape(N_ATOMS, D)
    pad = NPAD - N_PAIRS
    f_pad = jnp.pad(f_ij, ((0, pad), (0, 0)))
    rc_pad = jnp.pad(rcut_ij, (0, pad)).reshape(NPAD, 1)
    idxj32 = jnp.pad(idx_j.astype(jnp.int32), (0, pad))
    idxi32 = jnp.pad(idx_i.astype(jnp.int32), (0, pad))
    zeros = jnp.zeros((N_ROWS_PAD, D), jnp.float32)

    h, wij = _tc_a(x2, W_in, b_in.reshape(1, D), f_pad, rc_pad,
                   Wf1, bf1.reshape(1, D), Wf2, bf2.reshape(1, D))
    partials = _sc_gather_scatter(h, wij, idxj32, idxi32, zeros)
    out = _tc_b(partials, Wo1, bo1.reshape(1, D), Wo2, bo2.reshape(1, D))
    return out.reshape(1, N_ATOMS, D)


# speedup vs baseline: 2.8763x; 1.5111x over previous
import functools

import jax
import jax.numpy as jnp
from jax import lax
from jax.experimental import pallas as pl
from jax.experimental.pallas import tpu as pltpu
from jax.experimental.pallas import tpu_sc as plsc

N_ATOMS = 10000
D = 128
N_RBF = 20
N_PAIRS = 320000

NC, NS = 2, 16
NW = NC * NS
K = 80
PAIRS_PER_W = N_PAIRS // NW
CHUNKS = PAIRS_PER_W // K
N_ROWS_PAD = 10240
ROWS_PER_TILE = N_ROWS_PAD // NS

_LOG2 = 0.6931471805599453


def _ssp(t):
    return jnp.maximum(t, 0.0) + jnp.log1p(jnp.exp(-jnp.abs(t))) - _LOG2


P_BLK = 4000


def _tc_a_body(x_ref, W_in_ref, b_in_ref, f_ref, rc_ref, Wf1_ref, bf1_ref,
               Wf2_ref, bf2_ref, h_ref, wij_ref):
    @pl.when(pl.program_id(0) == 0)
    def _():
        h_ref[...] = lax.dot_general(
            x_ref[...], W_in_ref[...], (((1,), (1,)), ((), ())),
            preferred_element_type=jnp.float32) + b_in_ref[...]

    t = lax.dot_general(f_ref[...], Wf1_ref[...], (((1,), (1,)), ((), ())),
                        preferred_element_type=jnp.float32) + bf1_ref[...]
    w = lax.dot_general(_ssp(t), Wf2_ref[...], (((1,), (1,)), ((), ())),
                        preferred_element_type=jnp.float32) + bf2_ref[...]
    wij_ref[...] = w * rc_ref[...]


def _tc_a(x2, W_in, b_in, f_pad, rc_pad, Wf1, bf1, Wf2, bf2):
    nblk = N_PAIRS // P_BLK
    return pl.pallas_call(
        _tc_a_body,
        grid=(nblk,),
        in_specs=[
            pl.BlockSpec((N_ATOMS, D), lambda i: (0, 0)),
            pl.BlockSpec((D, D), lambda i: (0, 0)),
            pl.BlockSpec((1, D), lambda i: (0, 0)),
            pl.BlockSpec((P_BLK, N_RBF), lambda i: (i, 0)),
            pl.BlockSpec((P_BLK, 1), lambda i: (i, 0)),
            pl.BlockSpec((D, N_RBF), lambda i: (0, 0)),
            pl.BlockSpec((1, D), lambda i: (0, 0)),
            pl.BlockSpec((D, D), lambda i: (0, 0)),
            pl.BlockSpec((1, D), lambda i: (0, 0)),
        ],
        out_specs=[
            pl.BlockSpec((N_ATOMS, D), lambda i: (0, 0)),
            pl.BlockSpec((P_BLK, D), lambda i: (i, 0)),
        ],
        out_shape=[
            jax.ShapeDtypeStruct((N_ATOMS, D), jnp.float32),
            jax.ShapeDtypeStruct((N_PAIRS, D), jnp.float32),
        ],
    )(x2, W_in, b_in, f_pad, rc_pad, Wf1, bf1, Wf2, bf2)


def _sc_body(h_hbm, wij_hbm, idxj_hbm, idxi_hbm, zeros_hbm, out_hbm,
             idxj_v0, idxj_v1, idxi_v0, idxi_v1, rows_v0, rows_v1,
             wij_v0, wij_v1, agg_sh, semi0, semi1, semd0, semd1):
    c = lax.axis_index("c")
    s = lax.axis_index("s")
    wid = s * NC + c

    idxj_v = (idxj_v0, idxj_v1)
    idxi_v = (idxi_v0, idxi_v1)
    rows_v = (rows_v0, rows_v1)
    wij_v = (wij_v0, wij_v1)
    semi = (semi0, semi1)
    semd = (semd0, semd1)

    pltpu.sync_copy(zeros_hbm.at[pl.ds(s * ROWS_PER_TILE, ROWS_PER_TILE)],
                    agg_sh.at[pl.ds(s * ROWS_PER_TILE, ROWS_PER_TILE)])
    plsc.subcore_barrier()

    base0 = wid * PAIRS_PER_W

    def start_idx(g, b):
        base = base0 + g * K
        pltpu.async_copy(idxj_hbm.at[pl.ds(base, K)], idxj_v[b], semi[b])
        pltpu.async_copy(idxi_hbm.at[pl.ds(base, K)], idxi_v[b], semi[b])

    def start_dat(g, b):
        base = base0 + g * K
        pltpu.make_async_copy(idxj_hbm.at[pl.ds(base, K)], idxj_v[b], semi[b]).wait()
        pltpu.make_async_copy(idxi_hbm.at[pl.ds(base, K)], idxi_v[b], semi[b]).wait()
        pltpu.async_copy(h_hbm.at[idxj_v[b]], rows_v[b], semd[b])
        pltpu.async_copy(wij_hbm.at[pl.ds(base, K)], wij_v[b], semd[b])

    def finish(g, b):
        base = base0 + g * K
        pltpu.make_async_copy(h_hbm.at[idxj_v[b]], rows_v[b], semd[b]).wait()
        pltpu.make_async_copy(wij_hbm.at[pl.ds(base, K)], wij_v[b], semd[b]).wait()

        def mul_row(r, carry2):
            for col in range(D // 16):
                sl = pl.ds(col * 16, 16)
                rows_v[b][r, sl] = rows_v[b][r, sl] * wij_v[b][r, sl]
            return carry2

        lax.fori_loop(0, K, mul_row, 0, unroll=False)
        pltpu.sync_copy(rows_v[b], agg_sh.at[idxi_v[b]], add=True)

    start_idx(0, 0)
    start_idx(1, 1)
    start_dat(0, 0)

    def step(g, carry):
        b = lax.rem(g, 2)

        @pl.when(g + 1 < CHUNKS)
        def _():
            for bb in range(2):
                @pl.when(lax.rem(g + 1, 2) == bb)
                def _():
                    start_dat(g + 1, bb)

        for bb in range(2):
            @pl.when(b == bb)
            def _():
                finish(g, bb)

        @pl.when(g + 2 < CHUNKS)
        def _():
            for bb in range(2):
                @pl.when(lax.rem(g + 2, 2) == bb)
                def _():
                    start_idx(g + 2, bb)

        return carry

    lax.fori_loop(0, CHUNKS, step, 0, unroll=False)
    plsc.subcore_barrier()

    pltpu.sync_copy(agg_sh.at[pl.ds(s * ROWS_PER_TILE, ROWS_PER_TILE)],
                    out_hbm.at[c, pl.ds(s * ROWS_PER_TILE, ROWS_PER_TILE)])


_sc_gather_scatter = functools.partial(
    pl.kernel,
    mesh=plsc.VectorSubcoreMesh(core_axis_name="c", subcore_axis_name="s"),
    out_type=jax.ShapeDtypeStruct((NC, N_ROWS_PAD, D), jnp.float32),
    scratch_types=[
        pltpu.VMEM((K,), jnp.int32),
        pltpu.VMEM((K,), jnp.int32),
        pltpu.VMEM((K,), jnp.int32),
        pltpu.VMEM((K,), jnp.int32),
        pltpu.VMEM((K, D), jnp.float32),
        pltpu.VMEM((K, D), jnp.float32),
        pltpu.VMEM((K, D), jnp.float32),
        pltpu.VMEM((K, D), jnp.float32),
        pltpu.VMEM_SHARED((N_ROWS_PAD, D), jnp.float32),
        pltpu.SemaphoreType.DMA,
        pltpu.SemaphoreType.DMA,
        pltpu.SemaphoreType.DMA,
        pltpu.SemaphoreType.DMA,
    ],
)(_sc_body)


def _tc_b_body(p_ref, Wo1_ref, bo1_ref, Wo2_ref, bo2_ref, out_ref):
    agg = p_ref[0] + p_ref[1]
    t = lax.dot_general(agg, Wo1_ref[...], (((1,), (1,)), ((), ())),
                        preferred_element_type=jnp.float32) + bo1_ref[...]
    out_ref[...] = lax.dot_general(_ssp(t), Wo2_ref[...], (((1,), (1,)), ((), ())),
                                   preferred_element_type=jnp.float32) + bo2_ref[...]


def _tc_b(partials, Wo1, bo1, Wo2, bo2):
    return pl.pallas_call(
        _tc_b_body,
        grid=(1,),
        in_specs=[
            pl.BlockSpec((NC, N_ATOMS, D), lambda i: (0, 0, 0)),
            pl.BlockSpec((D, D), lambda i: (0, 0)),
            pl.BlockSpec((1, D), lambda i: (0, 0)),
            pl.BlockSpec((D, D), lambda i: (0, 0)),
            pl.BlockSpec((1, D), lambda i: (0, 0)),
        ],
        out_specs=pl.BlockSpec((N_ATOMS, D), lambda i: (0, 0)),
        out_shape=jax.ShapeDtypeStruct((N_ATOMS, D), jnp.float32),
    )(partials, Wo1, bo1, Wo2, bo2)


def kernel(x, f_ij, idx_i, idx_j, rcut_ij, W_in, b_in, Wf1, bf1, Wf2, bf2,
           Wo1, bo1, Wo2, bo2):
    x2 = x.reshape(N_ATOMS, D)
    rc2 = rcut_ij.reshape(N_PAIRS, 1)
    idxj32 = idx_j.astype(jnp.int32)
    idxi32 = idx_i.astype(jnp.int32)
    zeros = jnp.zeros((N_ROWS_PAD, D), jnp.float32)

    h, wij = _tc_a(x2, W_in, b_in.reshape(1, D), f_ij, rc2,
                   Wf1, bf1.reshape(1, D), Wf2, bf2.reshape(1, D))
    partials = _sc_gather_scatter(h, wij, idxj32, idxi32, zeros)
    out = _tc_b(partials, Wo1, bo1.reshape(1, D), Wo2, bo2.reshape(1, D))
    return out.reshape(1, N_ATOMS, D)


# speedup vs baseline: 3.7651x; 1.3090x over previous
import functools

import jax
import jax.numpy as jnp
from jax import lax
from jax.experimental import pallas as pl
from jax.experimental.pallas import tpu as pltpu
from jax.experimental.pallas import tpu_sc as plsc

N_ATOMS = 10000
D = 128
N_RBF = 20
N_PAIRS = 320000

NC, NS = 2, 16
NW = NC * NS
K = 80
PAIRS_PER_W = N_PAIRS // NW
CHUNKS = PAIRS_PER_W // K
N_ROWS_PAD = 10240
ROWS_PER_TILE = N_ROWS_PAD // NS

_LOG2 = 0.6931471805599453


def _ssp(t):
    return jnp.maximum(t, 0.0) + jnp.log1p(jnp.exp(-jnp.abs(t))) - _LOG2


P_BLK = 4096


def _tc_a_body(x_ref, W_in_ref, b_in_ref, f_ref, rc_ref, Wf1_ref, bf1_ref,
               Wf2_ref, bf2_ref, h_ref, wij_ref):
    @pl.when(pl.program_id(0) == 0)
    def _():
        h_ref[...] = lax.dot_general(
            x_ref[...], W_in_ref[...], (((1,), (1,)), ((), ())),
            preferred_element_type=jnp.float32) + b_in_ref[...]

    t = lax.dot_general(f_ref[...], Wf1_ref[...], (((1,), (1,)), ((), ())),
                        preferred_element_type=jnp.float32) + bf1_ref[...]
    w = lax.dot_general(_ssp(t), Wf2_ref[...], (((1,), (1,)), ((), ())),
                        preferred_element_type=jnp.float32) + bf2_ref[...]
    wij_ref[...] = w * rc_ref[...][:, None]


def _tc_a(x2, W_in, b_in, f_pad, rc_pad, Wf1, bf1, Wf2, bf2):
    nblk = -(-N_PAIRS // P_BLK)
    return pl.pallas_call(
        _tc_a_body,
        grid=(nblk,),
        in_specs=[
            pl.BlockSpec((N_ATOMS, D), lambda i: (0, 0)),
            pl.BlockSpec((D, D), lambda i: (0, 0)),
            pl.BlockSpec((1, D), lambda i: (0, 0)),
            pl.BlockSpec((P_BLK, N_RBF), lambda i: (i, 0)),
            pl.BlockSpec((P_BLK,), lambda i: (i,)),
            pl.BlockSpec((D, N_RBF), lambda i: (0, 0)),
            pl.BlockSpec((1, D), lambda i: (0, 0)),
            pl.BlockSpec((D, D), lambda i: (0, 0)),
            pl.BlockSpec((1, D), lambda i: (0, 0)),
        ],
        out_specs=[
            pl.BlockSpec((N_ATOMS, D), lambda i: (0, 0)),
            pl.BlockSpec((P_BLK, D), lambda i: (i, 0)),
        ],
        out_shape=[
            jax.ShapeDtypeStruct((N_ATOMS, D), jnp.float32),
            jax.ShapeDtypeStruct((N_PAIRS, D), jnp.float32),
        ],
    )(x2, W_in, b_in, f_pad, rc_pad, Wf1, bf1, Wf2, bf2)


def _sc_body(h_hbm, wij_hbm, idxj_hbm, idxi_hbm, zeros_hbm, out_hbm,
             idxj_v0, idxj_v1, idxi_v0, idxi_v1, rows_v0, rows_v1,
             wij_v0, wij_v1, agg_sh, semi0, semi1, semd0, semd1):
    c = lax.axis_index("c")
    s = lax.axis_index("s")
    wid = s * NC + c

    idxj_v = (idxj_v0, idxj_v1)
    idxi_v = (idxi_v0, idxi_v1)
    rows_v = (rows_v0, rows_v1)
    wij_v = (wij_v0, wij_v1)
    semi = (semi0, semi1)
    semd = (semd0, semd1)

    pltpu.sync_copy(zeros_hbm.at[pl.ds(s * ROWS_PER_TILE, ROWS_PER_TILE)],
                    agg_sh.at[pl.ds(s * ROWS_PER_TILE, ROWS_PER_TILE)])
    plsc.subcore_barrier()

    base0 = wid * PAIRS_PER_W

    def start_idx(g, b):
        base = base0 + g * K
        pltpu.async_copy(idxj_hbm.at[pl.ds(base, K)], idxj_v[b], semi[b])
        pltpu.async_copy(idxi_hbm.at[pl.ds(base, K)], idxi_v[b], semi[b])

    def start_dat(g, b):
        base = base0 + g * K
        pltpu.make_async_copy(idxj_hbm.at[pl.ds(base, K)], idxj_v[b], semi[b]).wait()
        pltpu.make_async_copy(idxi_hbm.at[pl.ds(base, K)], idxi_v[b], semi[b]).wait()
        pltpu.async_copy(h_hbm.at[idxj_v[b]], rows_v[b], semd[b])
        pltpu.async_copy(wij_hbm.at[pl.ds(base, K)], wij_v[b], semd[b])

    def finish(g, b):
        base = base0 + g * K
        pltpu.make_async_copy(h_hbm.at[idxj_v[b]], rows_v[b], semd[b]).wait()
        pltpu.make_async_copy(wij_hbm.at[pl.ds(base, K)], wij_v[b], semd[b]).wait()

        def mul_row(r, carry2):
            for col in range(D // 16):
                sl = pl.ds(col * 16, 16)
                rows_v[b][r, sl] = rows_v[b][r, sl] * wij_v[b][r, sl]
            return carry2

        lax.fori_loop(0, K, mul_row, 0, unroll=False)
        pltpu.sync_copy(rows_v[b], agg_sh.at[idxi_v[b]], add=True)

    start_idx(0, 0)
    start_idx(1, 1)
    start_dat(0, 0)

    def step(g, carry):
        b = lax.rem(g, 2)

        @pl.when(g + 1 < CHUNKS)
        def _():
            for bb in range(2):
                @pl.when(lax.rem(g + 1, 2) == bb)
                def _():
                    start_dat(g + 1, bb)

        for bb in range(2):
            @pl.when(b == bb)
            def _():
                finish(g, bb)

        @pl.when(g + 2 < CHUNKS)
        def _():
            for bb in range(2):
                @pl.when(lax.rem(g + 2, 2) == bb)
                def _():
                    start_idx(g + 2, bb)

        return carry

    lax.fori_loop(0, CHUNKS, step, 0, unroll=False)
    plsc.subcore_barrier()

    pltpu.sync_copy(agg_sh.at[pl.ds(s * ROWS_PER_TILE, ROWS_PER_TILE)],
                    out_hbm.at[c, pl.ds(s * ROWS_PER_TILE, ROWS_PER_TILE)])


_sc_gather_scatter = functools.partial(
    pl.kernel,
    mesh=plsc.VectorSubcoreMesh(core_axis_name="c", subcore_axis_name="s"),
    out_type=jax.ShapeDtypeStruct((NC, N_ROWS_PAD, D), jnp.float32),
    scratch_types=[
        pltpu.VMEM((K,), jnp.int32),
        pltpu.VMEM((K,), jnp.int32),
        pltpu.VMEM((K,), jnp.int32),
        pltpu.VMEM((K,), jnp.int32),
        pltpu.VMEM((K, D), jnp.float32),
        pltpu.VMEM((K, D), jnp.float32),
        pltpu.VMEM((K, D), jnp.float32),
        pltpu.VMEM((K, D), jnp.float32),
        pltpu.VMEM_SHARED((N_ROWS_PAD, D), jnp.float32),
        pltpu.SemaphoreType.DMA,
        pltpu.SemaphoreType.DMA,
        pltpu.SemaphoreType.DMA,
        pltpu.SemaphoreType.DMA,
    ],
)(_sc_body)


def _tc_b_body(p_ref, Wo1_ref, bo1_ref, Wo2_ref, bo2_ref, out_ref):
    agg = p_ref[0] + p_ref[1]
    t = lax.dot_general(agg, Wo1_ref[...], (((1,), (1,)), ((), ())),
                        preferred_element_type=jnp.float32) + bo1_ref[...]
    out_ref[...] = lax.dot_general(_ssp(t), Wo2_ref[...], (((1,), (1,)), ((), ())),
                                   preferred_element_type=jnp.float32) + bo2_ref[...]


def _tc_b(partials, Wo1, bo1, Wo2, bo2):
    return pl.pallas_call(
        _tc_b_body,
        grid=(1,),
        in_specs=[
            pl.BlockSpec((NC, N_ATOMS, D), lambda i: (0, 0, 0)),
            pl.BlockSpec((D, D), lambda i: (0, 0)),
            pl.BlockSpec((1, D), lambda i: (0, 0)),
            pl.BlockSpec((D, D), lambda i: (0, 0)),
            pl.BlockSpec((1, D), lambda i: (0, 0)),
        ],
        out_specs=pl.BlockSpec((N_ATOMS, D), lambda i: (0, 0)),
        out_shape=jax.ShapeDtypeStruct((N_ATOMS, D), jnp.float32),
    )(partials, Wo1, bo1, Wo2, bo2)


def kernel(x, f_ij, idx_i, idx_j, rcut_ij, W_in, b_in, Wf1, bf1, Wf2, bf2,
           Wo1, bo1, Wo2, bo2):
    x2 = x.reshape(N_ATOMS, D)
    rc2 = rcut_ij
    idxj32 = idx_j.astype(jnp.int32)
    idxi32 = idx_i.astype(jnp.int32)
    zeros = jnp.zeros((N_ROWS_PAD, D), jnp.float32)

    h, wij = _tc_a(x2, W_in, b_in.reshape(1, D), f_ij, rc2,
                   Wf1, bf1.reshape(1, D), Wf2, bf2.reshape(1, D))
    partials = _sc_gather_scatter(h, wij, idxj32, idxi32, zeros)
    out = _tc_b(partials, Wo1, bo1.reshape(1, D), Wo2, bo2.reshape(1, D))
    return out.reshape(1, N_ATOMS, D)


# speedup vs baseline: 4.6965x; 1.2474x over previous
import functools

import jax
import jax.numpy as jnp
from jax import lax
from jax.experimental import pallas as pl
from jax.experimental.pallas import tpu as pltpu
from jax.experimental.pallas import tpu_sc as plsc

N_ATOMS = 10000
D = 128
N_RBF = 20
N_PAIRS = 320000

NC, NS = 2, 16
NW = NC * NS
K = 80
PAIRS_PER_W = N_PAIRS // NW
CHUNKS = PAIRS_PER_W // K
N_ROWS_PAD = 10240
ROWS_PER_TILE = N_ROWS_PAD // NS

_LOG2 = 0.6931471805599453


def _ssp(t):
    return jnp.maximum(t, 0.0) + jnp.log1p(jnp.exp(-jnp.abs(t))) - _LOG2


P_BLK = 4096


def _tc_a_body(x_ref, W_in_ref, b_in_ref, f_ref, rc_ref, Wf1_ref, bf1_ref,
               Wf2_ref, bf2_ref, h_ref, wij_ref):
    @pl.when(pl.program_id(0) == 0)
    def _():
        h_ref[...] = lax.dot_general(
            x_ref[...], W_in_ref[...], (((1,), (1,)), ((), ())),
            preferred_element_type=jnp.float32) + b_in_ref[...]

    t = lax.dot_general(f_ref[...], Wf1_ref[...], (((0,), (1,)), ((), ())),
                        preferred_element_type=jnp.float32) + bf1_ref[...]
    w = lax.dot_general(_ssp(t), Wf2_ref[...], (((1,), (1,)), ((), ())),
                        preferred_element_type=jnp.float32) + bf2_ref[...]
    wij_ref[...] = w * rc_ref[...][:, None]


def _tc_a(x2, W_in, b_in, f_pad, rc_pad, Wf1, bf1, Wf2, bf2):
    nblk = -(-N_PAIRS // P_BLK)
    return pl.pallas_call(
        _tc_a_body,
        grid=(nblk,),
        in_specs=[
            pl.BlockSpec((N_ATOMS, D), lambda i: (0, 0)),
            pl.BlockSpec((D, D), lambda i: (0, 0)),
            pl.BlockSpec((1, D), lambda i: (0, 0)),
            pl.BlockSpec((N_RBF, P_BLK), lambda i: (0, i)),
            pl.BlockSpec((P_BLK,), lambda i: (i,)),
            pl.BlockSpec((D, N_RBF), lambda i: (0, 0)),
            pl.BlockSpec((1, D), lambda i: (0, 0)),
            pl.BlockSpec((D, D), lambda i: (0, 0)),
            pl.BlockSpec((1, D), lambda i: (0, 0)),
        ],
        out_specs=[
            pl.BlockSpec((N_ATOMS, D), lambda i: (0, 0)),
            pl.BlockSpec((P_BLK, D), lambda i: (i, 0)),
        ],
        out_shape=[
            jax.ShapeDtypeStruct((N_ATOMS, D), jnp.float32),
            jax.ShapeDtypeStruct((N_PAIRS, D), jnp.float32),
        ],
    )(x2, W_in, b_in, f_pad, rc_pad, Wf1, bf1, Wf2, bf2)


def _sc_body(h_hbm, wij_hbm, idxj_hbm, idxi_hbm, zeros_hbm, out_hbm,
             idxj_v0, idxj_v1, idxi_v0, idxi_v1, rows_v0, rows_v1,
             wij_v0, wij_v1, agg_sh, semi0, semi1, semd0, semd1):
    c = lax.axis_index("c")
    s = lax.axis_index("s")
    wid = s * NC + c

    idxj_v = (idxj_v0, idxj_v1)
    idxi_v = (idxi_v0, idxi_v1)
    rows_v = (rows_v0, rows_v1)
    wij_v = (wij_v0, wij_v1)
    semi = (semi0, semi1)
    semd = (semd0, semd1)

    pltpu.sync_copy(zeros_hbm.at[pl.ds(s * ROWS_PER_TILE, ROWS_PER_TILE)],
                    agg_sh.at[pl.ds(s * ROWS_PER_TILE, ROWS_PER_TILE)])
    plsc.subcore_barrier()

    base0 = wid * PAIRS_PER_W

    def start_idx(g, b):
        base = base0 + g * K
        pltpu.async_copy(idxj_hbm.at[pl.ds(base, K)], idxj_v[b], semi[b])
        pltpu.async_copy(idxi_hbm.at[pl.ds(base, K)], idxi_v[b], semi[b])

    def start_dat(g, b):
        base = base0 + g * K
        pltpu.make_async_copy(idxj_hbm.at[pl.ds(base, K)], idxj_v[b], semi[b]).wait()
        pltpu.make_async_copy(idxi_hbm.at[pl.ds(base, K)], idxi_v[b], semi[b]).wait()
        pltpu.async_copy(h_hbm.at[idxj_v[b]], rows_v[b], semd[b])
        pltpu.async_copy(wij_hbm.at[pl.ds(base, K)], wij_v[b], semd[b])

    def finish(g, b):
        base = base0 + g * K
        pltpu.make_async_copy(h_hbm.at[idxj_v[b]], rows_v[b], semd[b]).wait()
        pltpu.make_async_copy(wij_hbm.at[pl.ds(base, K)], wij_v[b], semd[b]).wait()

        def mul_row(r, carry2):
            for col in range(D // 16):
                sl = pl.ds(col * 16, 16)
                rows_v[b][r, sl] = rows_v[b][r, sl] * wij_v[b][r, sl]
            return carry2

        lax.fori_loop(0, K, mul_row, 0, unroll=False)
        pltpu.sync_copy(rows_v[b], agg_sh.at[idxi_v[b]], add=True)

    start_idx(0, 0)
    start_idx(1, 1)
    start_dat(0, 0)

    def step(g, carry):
        b = lax.rem(g, 2)

        @pl.when(g + 1 < CHUNKS)
        def _():
            for bb in range(2):
                @pl.when(lax.rem(g + 1, 2) == bb)
                def _():
                    start_dat(g + 1, bb)

        for bb in range(2):
            @pl.when(b == bb)
            def _():
                finish(g, bb)

        @pl.when(g + 2 < CHUNKS)
        def _():
            for bb in range(2):
                @pl.when(lax.rem(g + 2, 2) == bb)
                def _():
                    start_idx(g + 2, bb)

        return carry

    lax.fori_loop(0, CHUNKS, step, 0, unroll=False)
    plsc.subcore_barrier()

    pltpu.sync_copy(agg_sh.at[pl.ds(s * ROWS_PER_TILE, ROWS_PER_TILE)],
                    out_hbm.at[c, pl.ds(s * ROWS_PER_TILE, ROWS_PER_TILE)])


_sc_gather_scatter = functools.partial(
    pl.kernel,
    mesh=plsc.VectorSubcoreMesh(core_axis_name="c", subcore_axis_name="s"),
    out_type=jax.ShapeDtypeStruct((NC, N_ROWS_PAD, D), jnp.float32),
    scratch_types=[
        pltpu.VMEM((K,), jnp.int32),
        pltpu.VMEM((K,), jnp.int32),
        pltpu.VMEM((K,), jnp.int32),
        pltpu.VMEM((K,), jnp.int32),
        pltpu.VMEM((K, D), jnp.float32),
        pltpu.VMEM((K, D), jnp.float32),
        pltpu.VMEM((K, D), jnp.float32),
        pltpu.VMEM((K, D), jnp.float32),
        pltpu.VMEM_SHARED((N_ROWS_PAD, D), jnp.float32),
        pltpu.SemaphoreType.DMA,
        pltpu.SemaphoreType.DMA,
        pltpu.SemaphoreType.DMA,
        pltpu.SemaphoreType.DMA,
    ],
)(_sc_body)


def _tc_b_body(p_ref, Wo1_ref, bo1_ref, Wo2_ref, bo2_ref, out_ref):
    agg = p_ref[0] + p_ref[1]
    t = lax.dot_general(agg, Wo1_ref[...], (((1,), (1,)), ((), ())),
                        preferred_element_type=jnp.float32) + bo1_ref[...]
    out_ref[...] = lax.dot_general(_ssp(t), Wo2_ref[...], (((1,), (1,)), ((), ())),
                                   preferred_element_type=jnp.float32) + bo2_ref[...]


def _tc_b(partials, Wo1, bo1, Wo2, bo2):
    return pl.pallas_call(
        _tc_b_body,
        grid=(1,),
        in_specs=[
            pl.BlockSpec((NC, N_ATOMS, D), lambda i: (0, 0, 0)),
            pl.BlockSpec((D, D), lambda i: (0, 0)),
            pl.BlockSpec((1, D), lambda i: (0, 0)),
            pl.BlockSpec((D, D), lambda i: (0, 0)),
            pl.BlockSpec((1, D), lambda i: (0, 0)),
        ],
        out_specs=pl.BlockSpec((N_ATOMS, D), lambda i: (0, 0)),
        out_shape=jax.ShapeDtypeStruct((N_ATOMS, D), jnp.float32),
    )(partials, Wo1, bo1, Wo2, bo2)


def kernel(x, f_ij, idx_i, idx_j, rcut_ij, W_in, b_in, Wf1, bf1, Wf2, bf2,
           Wo1, bo1, Wo2, bo2):
    x2 = x.reshape(N_ATOMS, D)
    rc2 = rcut_ij
    idxj32 = idx_j.astype(jnp.int32)
    idxi32 = idx_i.astype(jnp.int32)
    zeros = jnp.zeros((N_ROWS_PAD, D), jnp.float32)

    h, wij = _tc_a(x2, W_in, b_in.reshape(1, D), f_ij.T, rc2,
                   Wf1, bf1.reshape(1, D), Wf2, bf2.reshape(1, D))
    partials = _sc_gather_scatter(h, wij, idxj32, idxi32, zeros)
    out = _tc_b(partials, Wo1, bo1.reshape(1, D), Wo2, bo2.reshape(1, D))
    return out.reshape(1, N_ATOMS, D)


# speedup vs baseline: 5.5028x; 1.1717x over previous
import functools

import jax
import jax.numpy as jnp
from jax import lax
from jax.experimental import pallas as pl
from jax.experimental.pallas import tpu as pltpu
from jax.experimental.pallas import tpu_sc as plsc

N_ATOMS = 10000
D = 128
N_RBF = 20
N_PAIRS = 320000

NC, NS = 2, 16
NW = NC * NS
K = 80
N_ROWS_PAD = 10240
ROWS_PER_TILE = N_ROWS_PAD // NS

P_BLK = 4096
SPLIT0 = 122880
SPLIT1 = N_PAIRS - SPLIT0

_LOG2 = 0.6931471805599453


def _ssp(t):
    return jnp.maximum(t, 0.0) + jnp.log1p(jnp.exp(-jnp.abs(t))) - _LOG2


def _tc_a_make(with_h, nblk, blk_off):
    def body_h(x_ref, W_in_ref, b_in_ref, f_ref, rc_ref, Wf1_ref, bf1_ref,
               Wf2_ref, bf2_ref, h_ref, wij_ref):
        @pl.when(pl.program_id(0) == 0)
        def _():
            h_ref[...] = lax.dot_general(
                x_ref[...], W_in_ref[...], (((1,), (1,)), ((), ())),
                preferred_element_type=jnp.float32) + b_in_ref[...]
        _filter(f_ref, rc_ref, Wf1_ref, bf1_ref, Wf2_ref, bf2_ref, wij_ref)

    def body(f_ref, rc_ref, Wf1_ref, bf1_ref, Wf2_ref, bf2_ref, wij_ref):
        _filter(f_ref, rc_ref, Wf1_ref, bf1_ref, Wf2_ref, bf2_ref, wij_ref)

    def _filter(f_ref, rc_ref, Wf1_ref, bf1_ref, Wf2_ref, bf2_ref, wij_ref):
        t = lax.dot_general(f_ref[...], Wf1_ref[...], (((0,), (1,)), ((), ())),
                            preferred_element_type=jnp.float32) + bf1_ref[...]
        w = lax.dot_general(_ssp(t), Wf2_ref[...], (((1,), (1,)), ((), ())),
                            preferred_element_type=jnp.float32) + bf2_ref[...]
        wij_ref[...] = w * rc_ref[...][:, None]

    f_spec = pl.BlockSpec((N_RBF, P_BLK), lambda i: (0, i + blk_off))
    rc_spec = pl.BlockSpec((P_BLK,), lambda i: (i + blk_off,))
    w_specs = [
        pl.BlockSpec((D, N_RBF), lambda i: (0, 0)),
        pl.BlockSpec((1, D), lambda i: (0, 0)),
        pl.BlockSpec((D, D), lambda i: (0, 0)),
        pl.BlockSpec((1, D), lambda i: (0, 0)),
    ]
    wij_spec = pl.BlockSpec((P_BLK, D), lambda i: (i, 0))
    wij_shape = jax.ShapeDtypeStruct((nblk * P_BLK, D), jnp.float32)
    if with_h:
        return pl.pallas_call(
            body_h,
            grid=(nblk,),
            in_specs=[
                pl.BlockSpec((N_ATOMS, D), lambda i: (0, 0)),
                pl.BlockSpec((D, D), lambda i: (0, 0)),
                pl.BlockSpec((1, D), lambda i: (0, 0)),
                f_spec, rc_spec, *w_specs,
            ],
            out_specs=[pl.BlockSpec((N_ATOMS, D), lambda i: (0, 0)), wij_spec],
            out_shape=[jax.ShapeDtypeStruct((N_ATOMS, D), jnp.float32), wij_shape],
        )
    return pl.pallas_call(
        body,
        grid=(nblk,),
        in_specs=[f_spec, rc_spec, *w_specs],
        out_specs=wij_spec,
        out_shape=wij_shape,
    )


_tc_a0 = _tc_a_make(True, SPLIT0 // P_BLK, 0)
_tc_a1 = _tc_a_make(False, -(-SPLIT1 // P_BLK), SPLIT0 // P_BLK)


def _sc_make(cpw, idx_base):
    def body(h_hbm, wij_hbm, idxj_hbm, idxi_hbm, init_hbm, out_hbm,
             idxj_v0, idxj_v1, idxi_v0, idxi_v1, rows_v0, rows_v1,
             wij_v0, wij_v1, agg_sh, semi0, semi1, semd0, semd1):
        c = lax.axis_index("c")
        s = lax.axis_index("s")
        wid = s * NC + c

        idxj_v = (idxj_v0, idxj_v1)
        idxi_v = (idxi_v0, idxi_v1)
        rows_v = (rows_v0, rows_v1)
        wij_v = (wij_v0, wij_v1)
        semi = (semi0, semi1)
        semd = (semd0, semd1)

        rsl = pl.ds(s * ROWS_PER_TILE, ROWS_PER_TILE)
        pltpu.sync_copy(init_hbm.at[c, rsl], agg_sh.at[rsl])
        plsc.subcore_barrier()

        wij_base = wid * (cpw * K)
        ib = idx_base + wij_base

        def start_idx(g, b):
            pltpu.async_copy(idxj_hbm.at[pl.ds(ib + g * K, K)], idxj_v[b], semi[b])
            pltpu.async_copy(idxi_hbm.at[pl.ds(ib + g * K, K)], idxi_v[b], semi[b])

        def start_dat(g, b):
            pltpu.make_async_copy(idxj_hbm.at[pl.ds(ib + g * K, K)], idxj_v[b], semi[b]).wait()
            pltpu.make_async_copy(idxi_hbm.at[pl.ds(ib + g * K, K)], idxi_v[b], semi[b]).wait()
            pltpu.async_copy(h_hbm.at[idxj_v[b]], rows_v[b], semd[b])
            pltpu.async_copy(wij_hbm.at[pl.ds(wij_base + g * K, K)], wij_v[b], semd[b])

        def finish(g, b):
            pltpu.make_async_copy(h_hbm.at[idxj_v[b]], rows_v[b], semd[b]).wait()
            pltpu.make_async_copy(wij_hbm.at[pl.ds(wij_base + g * K, K)], wij_v[b], semd[b]).wait()

            def mul_row(r, carry2):
                for col in range(D // 16):
                    sl = pl.ds(col * 16, 16)
                    rows_v[b][r, sl] = rows_v[b][r, sl] * wij_v[b][r, sl]
                return carry2

            lax.fori_loop(0, K, mul_row, 0, unroll=False)
            pltpu.sync_copy(rows_v[b], agg_sh.at[idxi_v[b]], add=True)

        start_idx(0, 0)
        start_idx(1, 1)
        start_dat(0, 0)

        def step(g, carry):
            b = lax.rem(g, 2)

            @pl.when(g + 1 < cpw)
            def _():
                for bb in range(2):
                    @pl.when(lax.rem(g + 1, 2) == bb)
                    def _():
                        start_dat(g + 1, bb)

            for bb in range(2):
                @pl.when(b == bb)
                def _():
                    finish(g, bb)

            @pl.when(g + 2 < cpw)
            def _():
                for bb in range(2):
                    @pl.when(lax.rem(g + 2, 2) == bb)
                    def _():
                        start_idx(g + 2, bb)

            return carry

        lax.fori_loop(0, cpw, step, 0, unroll=False)
        plsc.subcore_barrier()

        pltpu.sync_copy(agg_sh.at[rsl], out_hbm.at[c, rsl])

    return functools.partial(
        pl.kernel,
        mesh=plsc.VectorSubcoreMesh(core_axis_name="c", subcore_axis_name="s"),
        out_type=jax.ShapeDtypeStruct((NC, N_ROWS_PAD, D), jnp.float32),
        scratch_types=[
            pltpu.VMEM((K,), jnp.int32),
            pltpu.VMEM((K,), jnp.int32),
            pltpu.VMEM((K,), jnp.int32),
            pltpu.VMEM((K,), jnp.int32),
            pltpu.VMEM((K, D), jnp.float32),
            pltpu.VMEM((K, D), jnp.float32),
            pltpu.VMEM((K, D), jnp.float32),
            pltpu.VMEM((K, D), jnp.float32),
            pltpu.VMEM_SHARED((N_ROWS_PAD, D), jnp.float32),
            pltpu.SemaphoreType.DMA,
            pltpu.SemaphoreType.DMA,
            pltpu.SemaphoreType.DMA,
            pltpu.SemaphoreType.DMA,
        ],
    )(body)


_sc0 = _sc_make(SPLIT0 // (K * NW), 0)
_sc1 = _sc_make(SPLIT1 // (K * NW), SPLIT0)


def _tc_b_body(p_ref, Wo1_ref, bo1_ref, Wo2_ref, bo2_ref, out_ref):
    agg = p_ref[0] + p_ref[1]
    t = lax.dot_general(agg, Wo1_ref[...], (((1,), (1,)), ((), ())),
                        preferred_element_type=jnp.float32) + bo1_ref[...]
    out_ref[...] = lax.dot_general(_ssp(t), Wo2_ref[...], (((1,), (1,)), ((), ())),
                                   preferred_element_type=jnp.float32) + bo2_ref[...]


def _tc_b(partials, Wo1, bo1, Wo2, bo2):
    return pl.pallas_call(
        _tc_b_body,
        grid=(1,),
        in_specs=[
            pl.BlockSpec((NC, N_ATOMS, D), lambda i: (0, 0, 0)),
            pl.BlockSpec((D, D), lambda i: (0, 0)),
            pl.BlockSpec((1, D), lambda i: (0, 0)),
            pl.BlockSpec((D, D), lambda i: (0, 0)),
            pl.BlockSpec((1, D), lambda i: (0, 0)),
        ],
        out_specs=pl.BlockSpec((N_ATOMS, D), lambda i: (0, 0)),
        out_shape=jax.ShapeDtypeStruct((N_ATOMS, D), jnp.float32),
    )(partials, Wo1, bo1, Wo2, bo2)


def kernel(x, f_ij, idx_i, idx_j, rcut_ij, W_in, b_in, Wf1, bf1, Wf2, bf2,
           Wo1, bo1, Wo2, bo2):
    x2 = x.reshape(N_ATOMS, D)
    fT = f_ij.T
    idxj32 = idx_j.astype(jnp.int32)
    idxi32 = idx_i.astype(jnp.int32)
    zeros = jnp.zeros((NC, N_ROWS_PAD, D), jnp.float32)

    h, wij0 = _tc_a0(x2, W_in, b_in.reshape(1, D), fT, rcut_ij,
                     Wf1, bf1.reshape(1, D), Wf2, bf2.reshape(1, D))
    wij1 = _tc_a1(fT, rcut_ij, Wf1, bf1.reshape(1, D), Wf2, bf2.reshape(1, D))
    p0 = _sc0(h, wij0, idxj32, idxi32, zeros)
    p1 = _sc1(h, wij1, idxj32, idxi32, p0)
    out = _tc_b(p1, Wo1, bo1.reshape(1, D), Wo2, bo2.reshape(1, D))
    return out.reshape(1, N_ATOMS, D)


# speedup vs baseline: 5.5493x; 1.0084x over previous
import functools

import jax
import jax.numpy as jnp
import numpy as np
from jax import lax
from jax.experimental import pallas as pl
from jax.experimental.pallas import tpu as pltpu
from jax.experimental.pallas import tpu_sc as plsc

N_ATOMS = 10000
D = 128
N_RBF = 20
N_PAIRS = 320000

NC, NS = 2, 16
NW = NC * NS
K = 80
N_ROWS_PAD = 10240
ROWS_PER_TILE = N_ROWS_PAD // NS

P_BLK = 4096
SPLIT0 = 122880
SPLIT1 = N_PAIRS - SPLIT0

_LOG2 = 0.6931471805599453


def _ssp(t):
    return jnp.maximum(t, 0.0) + jnp.log1p(jnp.exp(-jnp.abs(t))) - _LOG2


def _tc_a_make(with_h, nblk, blk_off):
    def body_h(x_ref, W_in_ref, b_in_ref, f_ref, rc_ref, Wf1_ref, bf1_ref,
               Wf2_ref, bf2_ref, h_ref, wij_ref):
        @pl.when(pl.program_id(0) == 0)
        def _():
            h_ref[...] = lax.dot_general(
                x_ref[...], W_in_ref[...], (((1,), (1,)), ((), ())),
                preferred_element_type=jnp.float32) + b_in_ref[...]
        _filter(f_ref, rc_ref, Wf1_ref, bf1_ref, Wf2_ref, bf2_ref, wij_ref)

    def body(f_ref, rc_ref, Wf1_ref, bf1_ref, Wf2_ref, bf2_ref, wij_ref):
        _filter(f_ref, rc_ref, Wf1_ref, bf1_ref, Wf2_ref, bf2_ref, wij_ref)

    def _filter(f_ref, rc_ref, Wf1_ref, bf1_ref, Wf2_ref, bf2_ref, wij_ref):
        t = lax.dot_general(f_ref[...], Wf1_ref[...], (((0,), (1,)), ((), ())),
                            preferred_element_type=jnp.float32) + bf1_ref[...]
        w = lax.dot_general(_ssp(t), Wf2_ref[...], (((1,), (1,)), ((), ())),
                            preferred_element_type=jnp.float32) + bf2_ref[...]
        w = w * rc_ref[...][:, None]
        lo = lax.bitcast_convert_type(w[:, :64].astype(jnp.bfloat16), jnp.uint16)
        hi = lax.bitcast_convert_type(w[:, 64:].astype(jnp.bfloat16), jnp.uint16)
        wij_ref[...] = lax.bitwise_or(
            lo.astype(jnp.int32), lax.shift_left(hi.astype(jnp.int32), 16))

    f_spec = pl.BlockSpec((N_RBF, P_BLK), lambda i: (0, i + blk_off))
    rc_spec = pl.BlockSpec((P_BLK,), lambda i: (i + blk_off,))
    w_specs = [
        pl.BlockSpec((D, N_RBF), lambda i: (0, 0)),
        pl.BlockSpec((1, D), lambda i: (0, 0)),
        pl.BlockSpec((D, D), lambda i: (0, 0)),
        pl.BlockSpec((1, D), lambda i: (0, 0)),
    ]
    wij_spec = pl.BlockSpec((P_BLK, D // 2), lambda i: (i, 0))
    wij_shape = jax.ShapeDtypeStruct((nblk * P_BLK, D // 2), jnp.int32)
    if with_h:
        return pl.pallas_call(
            body_h,
            grid=(nblk,),
            in_specs=[
                pl.BlockSpec((N_ATOMS, D), lambda i: (0, 0)),
                pl.BlockSpec((D, D), lambda i: (0, 0)),
                pl.BlockSpec((1, D), lambda i: (0, 0)),
                f_spec, rc_spec, *w_specs,
            ],
            out_specs=[pl.BlockSpec((N_ATOMS, D), lambda i: (0, 0)), wij_spec],
            out_shape=[jax.ShapeDtypeStruct((N_ATOMS, D), jnp.float32), wij_shape],
        )
    return pl.pallas_call(
        body,
        grid=(nblk,),
        in_specs=[f_spec, rc_spec, *w_specs],
        out_specs=wij_spec,
        out_shape=wij_shape,
    )


_tc_a0 = _tc_a_make(True, SPLIT0 // P_BLK, 0)
_tc_a1 = _tc_a_make(False, -(-SPLIT1 // P_BLK), SPLIT0 // P_BLK)


def _sc_make(cpw, idx_base):
    def body(h_hbm, wij_hbm, idxj_hbm, idxi_hbm, init_hbm, out_hbm,
             idxj_v0, idxj_v1, idxi_v0, idxi_v1, rows_v0, rows_v1,
             wij_v0, wij_v1, agg_sh, semi0, semi1, semd0, semd1):
        c = lax.axis_index("c")
        s = lax.axis_index("s")
        wid = s * NC + c

        idxj_v = (idxj_v0, idxj_v1)
        idxi_v = (idxi_v0, idxi_v1)
        rows_v = (rows_v0, rows_v1)
        wij_v = (wij_v0, wij_v1)
        semi = (semi0, semi1)
        semd = (semd0, semd1)

        rsl = pl.ds(s * ROWS_PER_TILE, ROWS_PER_TILE)
        pltpu.sync_copy(init_hbm.at[c, rsl], agg_sh.at[rsl])
        plsc.subcore_barrier()

        wij_base = wid * (cpw * K)
        ib = idx_base + wij_base

        def start_idx(g, b):
            pltpu.async_copy(idxj_hbm.at[pl.ds(ib + g * K, K)], idxj_v[b], semi[b])
            pltpu.async_copy(idxi_hbm.at[pl.ds(ib + g * K, K)], idxi_v[b], semi[b])

        def start_dat(g, b):
            pltpu.make_async_copy(idxj_hbm.at[pl.ds(ib + g * K, K)], idxj_v[b], semi[b]).wait()
            pltpu.make_async_copy(idxi_hbm.at[pl.ds(ib + g * K, K)], idxi_v[b], semi[b]).wait()
            pltpu.async_copy(h_hbm.at[idxj_v[b]], rows_v[b], semd[b])
            pltpu.async_copy(wij_hbm.at[pl.ds(wij_base + g * K, K)], wij_v[b], semd[b])

        def finish(g, b):
            pltpu.make_async_copy(h_hbm.at[idxj_v[b]], rows_v[b], semd[b]).wait()
            pltpu.make_async_copy(wij_hbm.at[pl.ds(wij_base + g * K, K)], wij_v[b], semd[b]).wait()

            def mul_row(r, carry2):
                for q in range(D // 32):
                    wv = wij_v[b][r, pl.ds(16 * q, 16)]
                    lo = lax.bitcast_convert_type(
                        lax.shift_left(wv, jnp.full((16,), 16, jnp.int32)), jnp.float32)
                    hi = lax.bitcast_convert_type(
                        lax.bitwise_and(wv, jnp.full((16,), -65536, jnp.int32)), jnp.float32)
                    sl0 = pl.ds(16 * q, 16)
                    sl1 = pl.ds(64 + 16 * q, 16)
                    rows_v[b][r, sl0] = rows_v[b][r, sl0] * lo
                    rows_v[b][r, sl1] = rows_v[b][r, sl1] * hi
                return carry2

            lax.fori_loop(0, K, mul_row, 0, unroll=False)
            pltpu.sync_copy(rows_v[b], agg_sh.at[idxi_v[b]], add=True)

        start_idx(0, 0)
        start_idx(1, 1)
        start_dat(0, 0)

        def step(g, carry):
            b = lax.rem(g, 2)

            @pl.when(g + 1 < cpw)
            def _():
                for bb in range(2):
                    @pl.when(lax.rem(g + 1, 2) == bb)
                    def _():
                        start_dat(g + 1, bb)

            for bb in range(2):
                @pl.when(b == bb)
                def _():
                    finish(g, bb)

            @pl.when(g + 2 < cpw)
            def _():
                for bb in range(2):
                    @pl.when(lax.rem(g + 2, 2) == bb)
                    def _():
                        start_idx(g + 2, bb)

            return carry

        lax.fori_loop(0, cpw, step, 0, unroll=False)
        plsc.subcore_barrier()

        pltpu.sync_copy(agg_sh.at[rsl], out_hbm.at[c, rsl])

    return functools.partial(
        pl.kernel,
        mesh=plsc.VectorSubcoreMesh(core_axis_name="c", subcore_axis_name="s"),
        out_type=jax.ShapeDtypeStruct((NC, N_ROWS_PAD, D), jnp.float32),
        scratch_types=[
            pltpu.VMEM((K,), jnp.int32),
            pltpu.VMEM((K,), jnp.int32),
            pltpu.VMEM((K,), jnp.int32),
            pltpu.VMEM((K,), jnp.int32),
            pltpu.VMEM((K, D), jnp.float32),
            pltpu.VMEM((K, D), jnp.float32),
            pltpu.VMEM((K, D // 2), jnp.int32),
            pltpu.VMEM((K, D // 2), jnp.int32),
            pltpu.VMEM_SHARED((N_ROWS_PAD, D), jnp.float32),
            pltpu.SemaphoreType.DMA,
            pltpu.SemaphoreType.DMA,
            pltpu.SemaphoreType.DMA,
            pltpu.SemaphoreType.DMA,
        ],
    )(body)


_sc0 = _sc_make(SPLIT0 // (K * NW), 0)
_sc1 = _sc_make(SPLIT1 // (K * NW), SPLIT0)


def _tc_b_body(p_ref, Wo1_ref, bo1_ref, Wo2_ref, bo2_ref, out_ref):
    agg = p_ref[0] + p_ref[1]
    t = lax.dot_general(agg, Wo1_ref[...], (((1,), (1,)), ((), ())),
                        preferred_element_type=jnp.float32) + bo1_ref[...]
    out_ref[...] = lax.dot_general(_ssp(t), Wo2_ref[...], (((1,), (1,)), ((), ())),
                                   preferred_element_type=jnp.float32) + bo2_ref[...]


def _tc_b(partials, Wo1, bo1, Wo2, bo2):
    return pl.pallas_call(
        _tc_b_body,
        grid=(1,),
        in_specs=[
            pl.BlockSpec((NC, N_ATOMS, D), lambda i: (0, 0, 0)),
            pl.BlockSpec((D, D), lambda i: (0, 0)),
            pl.BlockSpec((1, D), lambda i: (0, 0)),
            pl.BlockSpec((D, D), lambda i: (0, 0)),
            pl.BlockSpec((1, D), lambda i: (0, 0)),
        ],
        out_specs=pl.BlockSpec((N_ATOMS, D), lambda i: (0, 0)),
        out_shape=jax.ShapeDtypeStruct((N_ATOMS, D), jnp.float32),
    )(partials, Wo1, bo1, Wo2, bo2)


def kernel(x, f_ij, idx_i, idx_j, rcut_ij, W_in, b_in, Wf1, bf1, Wf2, bf2,
           Wo1, bo1, Wo2, bo2):
    x2 = x.reshape(N_ATOMS, D)
    fT = f_ij.T
    idxj32 = idx_j.astype(jnp.int32)
    idxi32 = idx_i.astype(jnp.int32)
    zeros = jnp.zeros((NC, N_ROWS_PAD, D), jnp.float32)

    h, wij0 = _tc_a0(x2, W_in, b_in.reshape(1, D), fT, rcut_ij,
                     Wf1, bf1.reshape(1, D), Wf2, bf2.reshape(1, D))
    wij1 = _tc_a1(fT, rcut_ij, Wf1, bf1.reshape(1, D), Wf2, bf2.reshape(1, D))
    p0 = _sc0(h, wij0, idxj32, idxi32, zeros)
    p1 = _sc1(h, wij1, idxj32, idxi32, p0)
    out = _tc_b(p1, Wo1, bo1.reshape(1, D), Wo2, bo2.reshape(1, D))
    return out.reshape(1, N_ATOMS, D)
